# baseline (speedup 1.0000x reference)
import jax
import jax.numpy as jnp
from jax.experimental import pallas as pl

_H = 512
_W = 512
_C = 192
_B = 4096


def _cloak_block(o_ref, s_ref, out_ref):
    i = pl.program_id(0)
    o = o_ref[...]
    s = s_ref[...]
    dot = jnp.sum(o * s, axis=1, keepdims=True)
    n1 = jnp.sqrt(jnp.sum(o * o, axis=1, keepdims=True))
    n2 = jnp.sqrt(jnp.sum(s * s, axis=1, keepdims=True))
    eps = jnp.float32(1e-8)
    scores = dot / (jnp.maximum(n1, eps) * jnp.maximum(n2, eps))
    p = i * _B + jax.lax.broadcasted_iota(jnp.int32, (_B, 1), 0)
    mask = (
        (scores > 0.17)
        & (scores < 0.29)
        & (p >= _W)
        & ((p & (_W - 1)) != 0)
    )
    out_ref[...] = jnp.where(mask, s, o)


def kernel(original, styled):
    o2 = original.reshape(_H * _W, _C)
    s2 = styled.reshape(_H * _W, _C)
    out = pl.pallas_call(
        _cloak_block,
        grid=((_H * _W) // _B,),
        in_specs=[
            pl.BlockSpec((_B, _C), lambda i: (i, 0)),
            pl.BlockSpec((_B, _C), lambda i: (i, 0)),
        ],
        out_specs=pl.BlockSpec((_B, _C), lambda i: (i, 0)),
        out_shape=jax.ShapeDtypeStruct((_H * _W, _C), jnp.float32),
    )(o2, s2)
    return out.reshape(original.shape)
